# baseline (speedup 1.0000x reference)
import functools

import jax
import jax.numpy as jnp
from jax import lax
from jax.experimental import pallas as pl
from jax.experimental.pallas import tpu as pltpu
from jax.experimental.pallas import tpu_sc as plsc

N_USERS = 1000000
N_ITEMS = 100000
HIDDEN = 64
BATCH = 16384
FEAT_BITS = 17

NC = 2
NS = 16
NW = NC * NS
BPW = BATCH // NW
CHUNK = 128
NCHUNK = BPW // CHUNK

U1ROWS = (N_USERS + 127) // 128
I1ROWS = (N_ITEMS + 127) // 128

UHALF = N_USERS // 2
IHALF = N_ITEMS // 2

RLB = 5000


def _tc_relayout(W2, n_half):
    grid = (n_half // RLB,)
    nblocks = n_half // RLB

    def body(a_ref, b_ref, out_ref):
        out_ref[:, :HIDDEN] = a_ref[...]
        out_ref[:, HIDDEN:] = b_ref[...]

    return pl.pallas_call(
        body,
        grid=grid,
        in_specs=[
            pl.BlockSpec((RLB, HIDDEN), lambda i: (i, 0)),
            pl.BlockSpec((RLB, HIDDEN), lambda i: (i + nblocks, 0)),
        ],
        out_specs=pl.BlockSpec((RLB, 128), lambda i: (i, 0)),
        out_shape=jax.ShapeDtypeStruct((n_half, 128), jnp.float32),
        compiler_params=pltpu.CompilerParams(
            dimension_semantics=("parallel",)),
    )(W2, W2)


def _sc_gather_n(tables, indices, label):
    n = len(tables)
    mesh = plsc.VectorSubcoreMesh(core_axis_name="c", subcore_axis_name="s")
    row_t = jax.ShapeDtypeStruct((BATCH, 128), jnp.float32)

    @functools.partial(
        pl.kernel,
        mesh=mesh,
        out_type=(row_t,) * n,
        scratch_types=[pltpu.VMEM((BPW,), jnp.int32)] * n
          + [pltpu.VMEM((CHUNK, 128), jnp.float32)] * 4
          + [pltpu.SemaphoreType.DMA] * 8,
        name=label,
    )
    def k(*refs):
        tbl_hbm = refs[:n]
        idx_hbm = refs[n:2 * n]
        out_hbm = refs[2 * n:3 * n]
        idx_v = refs[3 * n:4 * n]
        bufs = refs[4 * n:4 * n + 4]
        gsems = refs[4 * n + 4:4 * n + 8]
        wsems = refs[4 * n + 8:4 * n + 12]
        wid = lax.axis_index("s") * NC + lax.axis_index("c")
        base = wid * BPW
        for t in range(n):
            pltpu.sync_copy(idx_hbm[t].at[pl.ds(base, BPW)], idx_v[t])

        streams = [(tbl_hbm[t], idx_v[t], out_hbm[t]) for t in range(n)]
        descs = [(streams[t], c) for c in range(NCHUNK) for t in range(n)]
        nd = len(descs)

        def fire_gather(kk, b):
            (tbl, idxr, _), c = descs[kk]
            return pltpu.async_copy(
                tbl.at[idxr.at[pl.ds(c * CHUNK, CHUNK)]], bufs[b], gsems[b])

        def fire_write(kk, b):
            (_, _, outr), c = descs[kk]
            return pltpu.async_copy(
                bufs[b], outr.at[pl.ds(base + c * CHUNK, CHUNK)], wsems[b])

        nb = min(4, nd)
        gc = [None] * nb
        wc = [None] * nb
        for kk in range(nb):
            gc[kk] = fire_gather(kk, kk)
        for kk in range(nd):
            b = kk % nb
            gc[b].wait()
            wc[b] = fire_write(kk, b)
            if kk + nb < nd:
                wc[b].wait()
                gc[b] = fire_gather(kk + nb, b)
        for kk in range(nd - nb, nd):
            wc[kk % nb].wait()

    return k(*tables, *indices)


def _tc_body(ui_ref, ii_ref, f0_ref, f1_ref, u2_ref, i2_ref, g1u_ref, g1i_ref,
             w2f0_ref, w2f1_ref, w1f_ref, bias_ref, out_ref):
    j = lax.broadcasted_iota(jnp.int32, (1, 128), 1)
    mask = jnp.where(j < FEAT_BITS,
                     jnp.left_shift(1, jnp.maximum(FEAT_BITS - 1 - j, 0)), 0)
    bits0 = (jnp.bitwise_and(f0_ref[...], mask) != 0).astype(jnp.float32)
    bits1 = (jnp.bitwise_and(f1_ref[...], mask) != 0).astype(jnp.float32)
    s0 = jnp.sum(bits0, axis=1, keepdims=True)
    s1 = jnp.sum(bits1, axis=1, keepdims=True)

    w1f = w1f_ref[...]
    fo0 = jnp.sum(bits0 * w1f[0:1, :], axis=1, keepdims=True) / s0
    fo1 = jnp.sum(bits1 * w1f[1:2, :], axis=1, keepdims=True) / s1

    e0 = jnp.dot(bits0, w2f0_ref[...],
                 preferred_element_type=jnp.float32,
                 precision=lax.Precision.HIGHEST) / s0
    e1 = jnp.dot(bits1, w2f1_ref[...],
                 preferred_element_type=jnp.float32,
                 precision=lax.Precision.HIGHEST) / s1

    ui = ui_ref[...]
    ii = ii_ref[...]

    w1u = jnp.sum(g1u_ref[...] * (jnp.bitwise_and(ui, 127) == j),
                  axis=1, keepdims=True)
    w1i = jnp.sum(g1i_ref[...] * (jnp.bitwise_and(ii, 127) == j),
                  axis=1, keepdims=True)

    urow = u2_ref[...]
    irow = i2_ref[...]
    u2 = jnp.where(ui < UHALF, urow[:, :HIDDEN], urow[:, HIDDEN:])
    i2 = jnp.where(jnp.bitwise_and(ii, 1) == 0,
                   irow[:, :HIDDEN], irow[:, HIDDEN:])

    ssum = u2 + i2 + e0 + e1
    diff = ssum * ssum - (u2 * u2 + i2 * i2 + e0 * e0 + e1 * e1)
    second = 0.5 * jnp.sum(diff, axis=1, keepdims=True)

    out_ref[...] = bias_ref[0, 0] + w1u + w1i + fo0 + fo1 + second


BB = 2048


def _tc_combine(ui, ii, f0, f1, u2, i2, g1u, g1i, W2f0p, W2f1p, w1f, bias2):
    grid = (BATCH // BB,)
    bspec = lambda bs: pl.BlockSpec(bs, lambda i: (i, 0))
    wspec = lambda bs: pl.BlockSpec(bs, lambda i: (0, 0))
    return pl.pallas_call(
        _tc_body,
        grid=grid,
        in_specs=[
            bspec((BB, 1)), bspec((BB, 1)), bspec((BB, 1)), bspec((BB, 1)),
            bspec((BB, 128)), bspec((BB, 128)),
            bspec((BB, 128)), bspec((BB, 128)),
            wspec((128, HIDDEN)), wspec((128, HIDDEN)),
            wspec((2, 128)), wspec((1, 1)),
        ],
        out_specs=bspec((BB, 1)),
        out_shape=jax.ShapeDtypeStruct((BATCH, 1), jnp.float32),
    )(ui, ii, f0, f1, u2, i2, g1u, g1i, W2f0p, W2f1p, w1f, bias2)


def kernel(x, bias, W1u, W1i, W1f0, W1f1, W2u, W2i, W2f0, W2f1):
    uidx = x[:, 0]
    iidx = x[:, 1]
    uw = uidx // 128
    iw = iidx // 128
    ur = jnp.where(uidx < UHALF, uidx, uidx - UHALF)
    ir = iidx // 2

    W1up = jnp.concatenate(
        [W1u.reshape(-1), jnp.zeros((U1ROWS * 128 - N_USERS,), jnp.float32)]
    ).reshape(U1ROWS, 128)
    W1ip = jnp.concatenate(
        [W1i.reshape(-1), jnp.zeros((I1ROWS * 128 - N_ITEMS,), jnp.float32)]
    ).reshape(I1ROWS, 128)

    W2u128 = _tc_relayout(W2u, UHALF)
    W2i128 = W2i.reshape(IHALF, 128)

    i2, g1u, g1i = _sc_gather_n(
        (W2i128, W1up, W1ip), (ir, uw, iw), "sc_small_gather")
    (u2,) = _sc_gather_n((W2u128,), (ur,), "sc_u_gather")

    pad = jnp.zeros((128 - FEAT_BITS, HIDDEN), jnp.float32)
    W2f0p = jnp.concatenate([W2f0, pad], axis=0)
    W2f1p = jnp.concatenate([W2f1, pad], axis=0)
    wpad = jnp.zeros((1, 128 - FEAT_BITS), jnp.float32)
    w1f = jnp.concatenate([
        jnp.concatenate([W1f0.T, wpad], axis=1),
        jnp.concatenate([W1f1.T, wpad], axis=1),
    ], axis=0)

    out = _tc_combine(
        x[:, 0:1], x[:, 1:2], x[:, 2:3], x[:, 3:4],
        u2, i2, g1u, g1i, W2f0p, W2f1p, w1f, bias.reshape(1, 1),
    )
    return out[:, 0]


# speedup vs baseline: 1.0535x; 1.0535x over previous
import functools

import jax
import jax.numpy as jnp
from jax import lax
from jax.experimental import pallas as pl
from jax.experimental.pallas import tpu as pltpu
from jax.experimental.pallas import tpu_sc as plsc

N_USERS = 1000000
N_ITEMS = 100000
HIDDEN = 64
BATCH = 16384
FEAT_BITS = 17

NC = 2
NS = 16
NW = NC * NS
BPW = BATCH // NW
CHUNK = 128
NCHUNK = BPW // CHUNK

U1ROWS = (N_USERS + 127) // 128
I1ROWS = (N_ITEMS + 127) // 128

UHALF = N_USERS // 2
IHALF = N_ITEMS // 2

RLB = 5000


def _tc_relayout(W2, n_half):
    grid = (n_half // RLB,)
    nblocks = n_half // RLB

    def body(a_ref, b_ref, out_ref):
        out_ref[:, :HIDDEN] = a_ref[...]
        out_ref[:, HIDDEN:] = b_ref[...]

    return pl.pallas_call(
        body,
        grid=grid,
        in_specs=[
            pl.BlockSpec((RLB, HIDDEN), lambda i: (i, 0)),
            pl.BlockSpec((RLB, HIDDEN), lambda i: (i + nblocks, 0)),
        ],
        out_specs=pl.BlockSpec((RLB, 128), lambda i: (i, 0)),
        out_shape=jax.ShapeDtypeStruct((n_half, 128), jnp.float32),
        compiler_params=pltpu.CompilerParams(
            dimension_semantics=("parallel",)),
    )(W2, W2)


def _sc_gather_n(tables, indices, label):
    n = len(tables)
    mesh = plsc.VectorSubcoreMesh(core_axis_name="c", subcore_axis_name="s")
    row_t = jax.ShapeDtypeStruct((BATCH, 128), jnp.float32)

    @functools.partial(
        pl.kernel,
        mesh=mesh,
        out_type=(row_t,) * n,
        scratch_types=[pltpu.VMEM((BPW,), jnp.int32)] * n
          + [pltpu.VMEM((CHUNK, 128), jnp.float32)] * 4
          + [pltpu.SemaphoreType.DMA] * 8,
        name=label,
        compiler_params=pltpu.CompilerParams(skip_device_barrier=True),
    )
    def k(*refs):
        tbl_hbm = refs[:n]
        idx_hbm = refs[n:2 * n]
        out_hbm = refs[2 * n:3 * n]
        idx_v = refs[3 * n:4 * n]
        bufs = refs[4 * n:4 * n + 4]
        gsems = refs[4 * n + 4:4 * n + 8]
        wsems = refs[4 * n + 8:4 * n + 12]
        wid = lax.axis_index("s") * NC + lax.axis_index("c")
        base = wid * BPW
        for t in range(n):
            pltpu.sync_copy(idx_hbm[t].at[pl.ds(base, BPW)], idx_v[t])

        streams = [(tbl_hbm[t], idx_v[t], out_hbm[t]) for t in range(n)]
        descs = [(streams[t], c) for c in range(NCHUNK) for t in range(n)]
        nd = len(descs)

        def fire_gather(kk, b):
            (tbl, idxr, _), c = descs[kk]
            return pltpu.async_copy(
                tbl.at[idxr.at[pl.ds(c * CHUNK, CHUNK)]], bufs[b], gsems[b])

        def fire_write(kk, b):
            (_, _, outr), c = descs[kk]
            return pltpu.async_copy(
                bufs[b], outr.at[pl.ds(base + c * CHUNK, CHUNK)], wsems[b])

        nb = min(4, nd)
        gc = [None] * nb
        wc = [None] * nb
        for kk in range(nb):
            gc[kk] = fire_gather(kk, kk)
        for kk in range(nd):
            b = kk % nb
            gc[b].wait()
            wc[b] = fire_write(kk, b)
            if kk + nb < nd:
                wc[b].wait()
                gc[b] = fire_gather(kk + nb, b)
        for kk in range(nd - nb, nd):
            wc[kk % nb].wait()

    return k(*tables, *indices)


def _tc_body(ui_ref, ii_ref, f0_ref, f1_ref, u2_ref, i2_ref, g1u_ref, g1i_ref,
             w2f0_ref, w2f1_ref, w1f_ref, bias_ref, out_ref):
    j = lax.broadcasted_iota(jnp.int32, (1, 128), 1)
    mask = jnp.where(j < FEAT_BITS,
                     jnp.left_shift(1, jnp.maximum(FEAT_BITS - 1 - j, 0)), 0)
    bits0 = (jnp.bitwise_and(f0_ref[...], mask) != 0).astype(jnp.float32)
    bits1 = (jnp.bitwise_and(f1_ref[...], mask) != 0).astype(jnp.float32)
    s0 = jnp.sum(bits0, axis=1, keepdims=True)
    s1 = jnp.sum(bits1, axis=1, keepdims=True)

    w1f = w1f_ref[...]
    fo0 = jnp.sum(bits0 * w1f[0:1, :], axis=1, keepdims=True) / s0
    fo1 = jnp.sum(bits1 * w1f[1:2, :], axis=1, keepdims=True) / s1

    e0 = jnp.dot(bits0, w2f0_ref[...],
                 preferred_element_type=jnp.float32,
                 precision=lax.Precision.HIGHEST) / s0
    e1 = jnp.dot(bits1, w2f1_ref[...],
                 preferred_element_type=jnp.float32,
                 precision=lax.Precision.HIGHEST) / s1

    ui = ui_ref[...]
    ii = ii_ref[...]

    w1u = jnp.sum(g1u_ref[...] * (jnp.bitwise_and(ui, 127) == j),
                  axis=1, keepdims=True)
    w1i = jnp.sum(g1i_ref[...] * (jnp.bitwise_and(ii, 127) == j),
                  axis=1, keepdims=True)

    urow = u2_ref[...]
    irow = i2_ref[...]
    u2 = jnp.where(jnp.bitwise_and(ui, 1) == 0,
                   urow[:, :HIDDEN], urow[:, HIDDEN:])
    i2 = jnp.where(jnp.bitwise_and(ii, 1) == 0,
                   irow[:, :HIDDEN], irow[:, HIDDEN:])

    ssum = u2 + i2 + e0 + e1
    diff = ssum * ssum - (u2 * u2 + i2 * i2 + e0 * e0 + e1 * e1)
    second = 0.5 * jnp.sum(diff, axis=1, keepdims=True)

    out_ref[...] = bias_ref[0, 0] + w1u + w1i + fo0 + fo1 + second


BB = 2048


def _tc_combine(ui, ii, f0, f1, u2, i2, g1u, g1i, W2f0p, W2f1p, w1f, bias2):
    grid = (BATCH // BB,)
    bspec = lambda bs: pl.BlockSpec(bs, lambda i: (i, 0))
    wspec = lambda bs: pl.BlockSpec(bs, lambda i: (0, 0))
    return pl.pallas_call(
        _tc_body,
        grid=grid,
        in_specs=[
            bspec((BB, 1)), bspec((BB, 1)), bspec((BB, 1)), bspec((BB, 1)),
            bspec((BB, 128)), bspec((BB, 128)),
            bspec((BB, 128)), bspec((BB, 128)),
            wspec((128, HIDDEN)), wspec((128, HIDDEN)),
            wspec((2, 128)), wspec((1, 1)),
        ],
        out_specs=bspec((BB, 1)),
        out_shape=jax.ShapeDtypeStruct((BATCH, 1), jnp.float32),
    )(ui, ii, f0, f1, u2, i2, g1u, g1i, W2f0p, W2f1p, w1f, bias2)


def kernel(x, bias, W1u, W1i, W1f0, W1f1, W2u, W2i, W2f0, W2f1):
    uidx = x[:, 0]
    iidx = x[:, 1]
    uw = uidx // 128
    iw = iidx // 128
    ur = uidx // 2
    ir = iidx // 2

    W1up = jnp.concatenate(
        [W1u.reshape(-1), jnp.zeros((U1ROWS * 128 - N_USERS,), jnp.float32)]
    ).reshape(U1ROWS, 128)
    W1ip = jnp.concatenate(
        [W1i.reshape(-1), jnp.zeros((I1ROWS * 128 - N_ITEMS,), jnp.float32)]
    ).reshape(I1ROWS, 128)

    W2u128 = W2u.reshape(UHALF, 128)
    W2i128 = W2i.reshape(IHALF, 128)

    i2, g1u, g1i = _sc_gather_n(
        (W2i128, W1up, W1ip), (ir, uw, iw), "sc_small_gather")
    (u2,) = _sc_gather_n((W2u128,), (ur,), "sc_u_gather")

    pad = jnp.zeros((128 - FEAT_BITS, HIDDEN), jnp.float32)
    W2f0p = jnp.concatenate([W2f0, pad], axis=0)
    W2f1p = jnp.concatenate([W2f1, pad], axis=0)
    wpad = jnp.zeros((1, 128 - FEAT_BITS), jnp.float32)
    w1f = jnp.concatenate([
        jnp.concatenate([W1f0.T, wpad], axis=1),
        jnp.concatenate([W1f1.T, wpad], axis=1),
    ], axis=0)

    out = _tc_combine(
        x[:, 0:1], x[:, 1:2], x[:, 2:3], x[:, 3:4],
        u2, i2, g1u, g1i, W2f0p, W2f1p, w1f, bias.reshape(1, 1),
    )
    return out[:, 0]
